# Initial kernel scaffold; baseline (speedup 1.0000x reference)
import jax
import jax.numpy as jnp
from jax.experimental import pallas as pl


def kernel(arm_loc, arm_conf, loc_data, conf_data, priors, targets):
    raise NotImplementedError("write your pallas kernel here")


# speedup vs baseline: 5.6045x; 5.6045x over previous
import functools

import jax
import jax.numpy as jnp
from jax import lax
from jax.experimental import pallas as pl
from jax.experimental.pallas import tpu as pltpu
from jax.experimental.pallas import tpu_sc as plsc

_B = 32
_P = 6375
_PP = 6384
_C = 21
_T = 10
_CW = 912
_NCH = _PP // _CW
_GPC = _CW // 16
_GROUPS = _PP // 16

_VAR0 = 0.1
_VAR1 = 0.2
_REFINE_T = 0.99
_NEGPOS = 3
_LN2 = 0.6931471805599453
_NEG_INF = -3.0e38


def _log_pos(x):
    bits = plsc.bitcast(x, jnp.int32)
    e = (bits >> 23) - 127
    m = plsc.bitcast((bits & 0x007FFFFF) | 0x3F800000, jnp.float32)
    big = m > 1.4142135623730951
    m = jnp.where(big, m * 0.5, m)
    e = jnp.where(big, e + 1, e)
    t = (m - 1.0) / (m + 1.0)
    t2 = t * t
    p = 1.0 / 7.0 + t2 * (1.0 / 9.0)
    p = 1.0 / 5.0 + t2 * p
    p = 1.0 / 3.0 + t2 * p
    p = 2.0 * t * (1.0 + t2 * p)
    return e.astype(jnp.float32) * _LN2 + p


def _sc_body(pri_h, arm_h, ac0_h, conf_h, locd_h, tgt_h, out_h,
             pri_v, arm_v, ac0_v, tgt_v, bto_v, bti_v, s_v, conf_c, loc_c,
             res_v):
    b = lax.axis_index("s") * 2 + lax.axis_index("c")

    pltpu.sync_copy(pri_h, pri_v)
    pltpu.sync_copy(arm_h.at[pl.ds(b * (_PP * 4), _PP * 4)], arm_v)
    pltpu.sync_copy(ac0_h.at[pl.ds(b * _PP, _PP)], ac0_v)
    pltpu.sync_copy(tgt_h.at[pl.ds(b * 96, 96)], tgt_v)

    lane16 = lax.iota(jnp.int32, 16)

    trow = [tgt_v[pl.ds(t * 8, 16)] for t in range(_T)]
    tx0 = [trow[t][0] for t in range(_T)]
    ty0 = [trow[t][1] for t in range(_T)]
    tx1 = [trow[t][2] for t in range(_T)]
    ty1 = [trow[t][3] for t in range(_T)]
    tarea = [(tx1[t] - tx0[t]) * (ty1[t] - ty0[t]) for t in range(_T)]

    def p1(g, carry):
        idxL = lane16 + g * 16
        i4 = idxL * 4
        al = [plsc.load_gather(arm_v, [i4 + j]) for j in range(4)]
        pr = [plsc.load_gather(pri_v, [i4 + j]) for j in range(4)]
        cx = pr[0] + al[0] * (_VAR0 * pr[2])
        cy = pr[1] + al[1] * (_VAR0 * pr[3])
        w = pr[2] * jnp.exp(al[2] * _VAR1)
        h = pr[3] * jnp.exp(al[3] * _VAR1)
        x0 = cx - w * 0.5
        y0 = cy - h * 0.5
        x1 = cx + w * 0.5
        y1 = cy + h * 0.5
        rw = x1 - x0
        rh = y1 - y0
        plsc.store_scatter(arm_v, [i4 + 0], (x0 + x1) * 0.5)
        plsc.store_scatter(arm_v, [i4 + 1], (y0 + y1) * 0.5)
        plsc.store_scatter(arm_v, [i4 + 2], rw)
        plsc.store_scatter(arm_v, [i4 + 3], rh)
        area_p = rw * rh
        ig = ac0_v[pl.ds(g * 16, 16)] > _REFINE_T
        btoL = jnp.full((16,), _NEG_INF, jnp.float32)
        btiL = jnp.zeros((16,), jnp.int32)
        new = []
        for t in range(_T):
            ix = jnp.maximum(
                jnp.minimum(tx1[t], x1) - jnp.maximum(tx0[t], x0), 0.0)
            iy = jnp.maximum(
                jnp.minimum(ty1[t], y1) - jnp.maximum(ty0[t], y0), 0.0)
            inter = ix * iy
            ov = inter / (tarea[t] + area_p - inter)
            ov = jnp.where(ig, -1.0, ov)
            updL = ov > btoL
            btoL = jnp.where(updL, ov, btoL)
            btiL = jnp.where(updL, t, btiL)
            bv, bi = carry[2 * t], carry[2 * t + 1]
            updT = ov > bv
            new.append(jnp.where(updT, ov, bv))
            new.append(jnp.where(updT, idxL, bi))
        bto_v[pl.ds(g * 16, 16)] = btoL
        bti_v[pl.ds(g * 16, 16)] = btiL
        return tuple(new)

    init = tuple(
        jnp.full((16,), _NEG_INF, jnp.float32) if i % 2 == 0
        else jnp.zeros((16,), jnp.int32)
        for i in range(2 * _T))
    best = lax.fori_loop(0, _GROUPS, p1, init)

    bps = []
    for t in range(_T):
        bv, bi = best[2 * t], best[2 * t + 1]
        mx = jnp.max(bv)
        bps.append(jnp.min(jnp.where(bv == mx, bi, jnp.int32(1 << 30))))

    def make_p2(ch):
        def p2(g, carry):
            ll_a, lcp_a, np_a = carry
            gg = ch * _GPC + g
            ds = pl.ds(gg * 16, 16)
            idxL = lane16 + gg * 16
            bto = bto_v[ds]
            bti = bti_v[ds]
            for t in range(_T):
                hit = idxL == bps[t]
                bto = jnp.where(hit, 2.0, bto)
                bti = jnp.where(hit, t, bti)
            ig = ac0_v[ds] > _REFINE_T
            i8 = bti * 8
            labf = plsc.load_gather(tgt_v, [i8 + 4])
            confc = labf.astype(jnp.int32) + 1
            confc = jnp.where(bto < 0.5, 0, confc)
            confc = jnp.where(ig, -1, confc)
            pos = confc > 0
            ct0 = jnp.maximum(confc, 0)
            iloc = (lane16 + g * 16) * _C
            rows = [plsc.load_gather(conf_c, [iloc + cc]) for cc in range(_C)]
            mx = rows[0]
            for cc in range(1, _C):
                mx = jnp.maximum(mx, rows[cc])
            se = jnp.exp(rows[0] - mx)
            for cc in range(1, _C):
                se = se + jnp.exp(rows[cc] - mx)
            lse = _log_pos(se) + mx
            gold = plsc.load_gather(conf_c, [iloc + ct0])
            ce = lse - gold
            sv = jnp.where(pos | ig, 0.0, ce)
            sv = jnp.where(idxL >= _P, -1.0, sv)
            s_v[ds] = sv
            lcp_a = lcp_a + jnp.where(pos, ce, 0.0)
            np_a = np_a + jnp.where(pos, 1, 0)
            i4 = idxL * 4
            rcx = plsc.load_gather(arm_v, [i4 + 0])
            rcy = plsc.load_gather(arm_v, [i4 + 1])
            rw = plsc.load_gather(arm_v, [i4 + 2])
            rh = plsc.load_gather(arm_v, [i4 + 3])
            mt = [plsc.load_gather(tgt_v, [i8 + j]) for j in range(4)]
            l0 = ((mt[0] + mt[2]) * 0.5 - rcx) / (_VAR0 * rw)
            l1 = ((mt[1] + mt[3]) * 0.5 - rcy) / (_VAR0 * rh)
            l2 = _log_pos((mt[2] - mt[0]) / rw) * (1.0 / _VAR1)
            l3 = _log_pos((mt[3] - mt[1]) / rh) * (1.0 / _VAR1)
            il4 = (lane16 + g * 16) * 4
            for j, lj in enumerate((l0, l1, l2, l3)):
                d = plsc.load_gather(loc_c, [il4 + j]) - lj
                ad = jnp.abs(d)
                sl = jnp.where(ad < 1.0, 0.5 * d * d, ad - 0.5)
                ll_a = ll_a + jnp.where(pos, sl, 0.0)
            return ll_a, lcp_a, np_a
        return p2

    zf = jnp.zeros((16,), jnp.float32)
    acc = (zf, zf, jnp.zeros((16,), jnp.int32))
    for ch in range(_NCH):
        pltpu.sync_copy(
            conf_h.at[pl.ds((b * _NCH + ch) * (_CW * _C), _CW * _C)], conf_c)
        pltpu.sync_copy(
            locd_h.at[pl.ds((b * _NCH + ch) * (_CW * 4), _CW * 4)], loc_c)
        acc = lax.fori_loop(0, _GPC, make_p2(ch), acc)

    ll = jnp.sum(acc[0])
    lc_pos = jnp.sum(acc[1])
    npos = jnp.sum(acc[2])

    k = jnp.minimum(npos * _NEGPOS, _P - 1)

    def count_ge(cand):
        def cb(g, cnt):
            sb = plsc.bitcast(s_v[pl.ds(g * 16, 16)], jnp.int32)
            return cnt + jnp.sum((sb >= cand).astype(jnp.int32))
        return lax.fori_loop(0, _GROUPS, cb, jnp.int32(0))

    m = jnp.int32(0)
    for bit in range(30, -1, -1):
        cand = m | jnp.int32(1 << bit)
        m = jnp.where(count_ge(cand) >= k, cand, m)

    def cgt_ceq(g, c2):
        cg, ceq = c2
        sb = plsc.bitcast(s_v[pl.ds(g * 16, 16)], jnp.int32)
        return (cg + jnp.sum((sb > m).astype(jnp.int32)),
                ceq + jnp.sum((sb == m).astype(jnp.int32)))

    cnt_gt, cnt_eq = lax.fori_loop(0, _GROUPS, cgt_ceq,
                                   (jnp.int32(0), jnp.int32(0)))
    need_eq = k - cnt_gt

    def idx_search(_):
        def count_lt(cand):
            def cb(g, cnt):
                ds = pl.ds(g * 16, 16)
                sb = plsc.bitcast(s_v[ds], jnp.int32)
                idxL = lane16 + g * 16
                hit = (sb == m) & (idxL < cand)
                return cnt + jnp.sum(hit.astype(jnp.int32))
            return lax.fori_loop(0, _GROUPS, cb, jnp.int32(0))
        jt = jnp.int32(0)
        for bit in range(12, -1, -1):
            cand = jt | jnp.int32(1 << bit)
            jt = jnp.where(count_lt(cand) <= need_eq, cand, jt)
        return jt

    def idx_trivial(_):
        return jnp.where(need_eq > 0, jnp.int32(_PP), jnp.int32(0))

    partial_ties = (need_eq > 0) & (need_eq < cnt_eq)
    jthr = lax.cond(partial_ties, idx_search, idx_trivial, 0)

    v = plsc.bitcast(jnp.full((16,), m, jnp.int32), jnp.float32)

    def negsum(g, a):
        ds = pl.ds(g * 16, 16)
        sv = s_v[ds]
        idxL = lane16 + g * 16
        sel = (sv > v) | ((sv == v) & (idxL < jthr))
        return a + jnp.where(sel, sv, 0.0)

    lc = lc_pos + jnp.sum(lax.fori_loop(0, _GROUPS, negsum, zf))

    res = jnp.where(lane16 == 0, ll, 0.0)
    res = jnp.where(lane16 == 1, lc, res)
    res = jnp.where(lane16 == 2, npos.astype(jnp.float32), res)
    res_v[...] = res
    pltpu.sync_copy(res_v, out_h.at[pl.ds(b * 16, 16)])


@functools.partial(
    pl.kernel,
    out_type=jax.ShapeDtypeStruct((_B * 16,), jnp.float32),
    mesh=plsc.VectorSubcoreMesh(core_axis_name="c", subcore_axis_name="s"),
    compiler_params=pltpu.CompilerParams(needs_layout_passes=False),
    scratch_types=[
        pltpu.VMEM((_PP * 4,), jnp.float32),
        pltpu.VMEM((_PP * 4,), jnp.float32),
        pltpu.VMEM((_PP,), jnp.float32),
        pltpu.VMEM((_T * 8 + 16,), jnp.float32),
        pltpu.VMEM((_PP,), jnp.float32),
        pltpu.VMEM((_PP,), jnp.int32),
        pltpu.VMEM((_PP,), jnp.float32),
        pltpu.VMEM((_CW * _C,), jnp.float32),
        pltpu.VMEM((_CW * 4,), jnp.float32),
        pltpu.VMEM((16,), jnp.float32),
    ],
)
def _sc_kernel(*args):
    _sc_body(*args)


def kernel(arm_loc, arm_conf, loc_data, conf_data, priors, targets):
    padp = _PP - _P
    pri = jnp.pad(priors, ((0, padp), (0, 0))).reshape(_PP * 4)
    arm = jnp.pad(arm_loc, ((0, 0), (0, padp), (0, 0))).reshape(_B * _PP * 4)
    ac0 = jnp.pad(arm_conf[:, :, 0], ((0, 0), (0, padp)),
                  constant_values=2.0).reshape(_B * _PP)
    conf = jnp.pad(conf_data, ((0, 0), (0, padp), (0, 0)))
    conf = conf.reshape(_B * _NCH * _CW * _C)
    locd = jnp.pad(loc_data, ((0, 0), (0, padp), (0, 0)))
    locd = locd.reshape(_B * _NCH * _CW * 4)
    tgt = jnp.pad(targets, ((0, 0), (0, 0), (0, 3))).reshape(_B, _T * 8)
    tgt = jnp.pad(tgt, ((0, 0), (0, 16))).reshape(_B * 96)

    out = _sc_kernel(pri, arm, ac0, conf, locd, tgt).reshape(_B, 16)
    n = jnp.sum(out[:, 2])
    return jnp.sum(out[:, 0]) / n, jnp.sum(out[:, 1]) / n


# speedup vs baseline: 5.7301x; 1.0224x over previous
import functools

import jax
import jax.numpy as jnp
from jax import lax
from jax.experimental import pallas as pl
from jax.experimental.pallas import tpu as pltpu
from jax.experimental.pallas import tpu_sc as plsc

_B = 32
_P = 6375
_PP = 6384
_C = 21
_T = 10
_CW = 912
_NCH = _PP // _CW
_GPC = _CW // 16
_GROUPS = _PP // 16

_VAR0 = 0.1
_VAR1 = 0.2
_REFINE_T = 0.99
_NEGPOS = 3
_LN2 = 0.6931471805599453
_NEG_INF = -3.0e38


def _log_pos(x):
    bits = plsc.bitcast(x, jnp.int32)
    e = (bits >> 23) - 127
    m = plsc.bitcast((bits & 0x007FFFFF) | 0x3F800000, jnp.float32)
    big = m > 1.4142135623730951
    m = jnp.where(big, m * 0.5, m)
    e = jnp.where(big, e + 1, e)
    t = (m - 1.0) / (m + 1.0)
    t2 = t * t
    p = 1.0 / 7.0 + t2 * (1.0 / 9.0)
    p = 1.0 / 5.0 + t2 * p
    p = 1.0 / 3.0 + t2 * p
    p = 2.0 * t * (1.0 + t2 * p)
    return e.astype(jnp.float32) * _LN2 + p


def _sc_body(pri_h, arm_h, ac0_h, conf_h, locd_h, tgt_h, out_h,
             pri_v, arm_v, ac0_v, tgt_v, bto_v, bti_v, s_v, conf_c, loc_c,
             res_v, sems):
    b = lax.axis_index("s") * 2 + lax.axis_index("c")

    def conf_dma(ch, buf):
        return pltpu.async_copy(
            conf_h.at[pl.ds((b * _NCH + ch) * (_CW * _C), _CW * _C)],
            conf_c[buf], sems[4 + buf])

    def loc_dma(ch, buf):
        return pltpu.async_copy(
            locd_h.at[pl.ds((b * _NCH + ch) * (_CW * 4), _CW * 4)],
            loc_c[buf], sems[6 + buf])

    h_pri = pltpu.async_copy(pri_h, pri_v, sems[0])
    h_arm = pltpu.async_copy(
        arm_h.at[pl.ds(b * (_PP * 4), _PP * 4)], arm_v, sems[1])
    h_ac0 = pltpu.async_copy(ac0_h.at[pl.ds(b * _PP, _PP)], ac0_v, sems[2])
    h_tgt = pltpu.async_copy(tgt_h.at[pl.ds(b * 96, 96)], tgt_v, sems[3])
    h_conf = conf_dma(0, 0)
    h_loc = loc_dma(0, 0)
    h_pri.wait()
    h_arm.wait()
    h_ac0.wait()
    h_tgt.wait()

    lane16 = lax.iota(jnp.int32, 16)

    trow = [tgt_v[pl.ds(t * 8, 16)] for t in range(_T)]
    tx0 = [trow[t][0] for t in range(_T)]
    ty0 = [trow[t][1] for t in range(_T)]
    tx1 = [trow[t][2] for t in range(_T)]
    ty1 = [trow[t][3] for t in range(_T)]
    tarea = [(tx1[t] - tx0[t]) * (ty1[t] - ty0[t]) for t in range(_T)]

    def p1(g, carry):
        idxL = lane16 + g * 16
        i4 = idxL * 4
        al = [plsc.load_gather(arm_v, [i4 + j]) for j in range(4)]
        pr = [plsc.load_gather(pri_v, [i4 + j]) for j in range(4)]
        cx = pr[0] + al[0] * (_VAR0 * pr[2])
        cy = pr[1] + al[1] * (_VAR0 * pr[3])
        w = pr[2] * jnp.exp(al[2] * _VAR1)
        h = pr[3] * jnp.exp(al[3] * _VAR1)
        x0 = cx - w * 0.5
        y0 = cy - h * 0.5
        x1 = cx + w * 0.5
        y1 = cy + h * 0.5
        rw = x1 - x0
        rh = y1 - y0
        plsc.store_scatter(arm_v, [i4 + 0], (x0 + x1) * 0.5)
        plsc.store_scatter(arm_v, [i4 + 1], (y0 + y1) * 0.5)
        plsc.store_scatter(arm_v, [i4 + 2], rw)
        plsc.store_scatter(arm_v, [i4 + 3], rh)
        area_p = rw * rh
        ig = ac0_v[pl.ds(g * 16, 16)] > _REFINE_T
        btoL = jnp.full((16,), _NEG_INF, jnp.float32)
        btiL = jnp.zeros((16,), jnp.int32)
        new = []
        for t in range(_T):
            ix = jnp.maximum(
                jnp.minimum(tx1[t], x1) - jnp.maximum(tx0[t], x0), 0.0)
            iy = jnp.maximum(
                jnp.minimum(ty1[t], y1) - jnp.maximum(ty0[t], y0), 0.0)
            inter = ix * iy
            ov = inter / (tarea[t] + area_p - inter)
            ov = jnp.where(ig, -1.0, ov)
            updL = ov > btoL
            btoL = jnp.where(updL, ov, btoL)
            btiL = jnp.where(updL, t, btiL)
            bv, bi = carry[2 * t], carry[2 * t + 1]
            updT = ov > bv
            new.append(jnp.where(updT, ov, bv))
            new.append(jnp.where(updT, idxL, bi))
        bto_v[pl.ds(g * 16, 16)] = btoL
        bti_v[pl.ds(g * 16, 16)] = btiL
        return tuple(new)

    init = tuple(
        jnp.full((16,), _NEG_INF, jnp.float32) if i % 2 == 0
        else jnp.zeros((16,), jnp.int32)
        for i in range(2 * _T))
    best = lax.fori_loop(0, _GROUPS, p1, init)

    bps = []
    for t in range(_T):
        bv, bi = best[2 * t], best[2 * t + 1]
        mx = jnp.max(bv)
        bps.append(jnp.min(jnp.where(bv == mx, bi, jnp.int32(1 << 30))))

    def make_p2(ch, buf):
        def p2(g, carry):
            ll_a, lcp_a, np_a = carry
            gg = ch * _GPC + g
            ds = pl.ds(gg * 16, 16)
            idxL = lane16 + gg * 16
            bto = bto_v[ds]
            bti = bti_v[ds]
            for t in range(_T):
                hit = idxL == bps[t]
                bto = jnp.where(hit, 2.0, bto)
                bti = jnp.where(hit, t, bti)
            ig = ac0_v[ds] > _REFINE_T
            i8 = bti * 8
            labf = plsc.load_gather(tgt_v, [i8 + 4])
            confc = labf.astype(jnp.int32) + 1
            confc = jnp.where(bto < 0.5, 0, confc)
            confc = jnp.where(ig, -1, confc)
            pos = confc > 0
            ct0 = jnp.maximum(confc, 0)
            iloc = (lane16 + g * 16) * _C
            rows = [plsc.load_gather(conf_c[buf], [iloc + cc])
                    for cc in range(_C)]
            mx = rows[0]
            for cc in range(1, _C):
                mx = jnp.maximum(mx, rows[cc])
            se = jnp.exp(rows[0] - mx)
            for cc in range(1, _C):
                se = se + jnp.exp(rows[cc] - mx)
            lse = _log_pos(se) + mx
            gold = plsc.load_gather(conf_c[buf], [iloc + ct0])
            ce = lse - gold
            sv = jnp.where(pos | ig, 0.0, ce)
            sv = jnp.where(idxL >= _P, -1.0, sv)
            s_v[ds] = sv
            lcp_a = lcp_a + jnp.where(pos, ce, 0.0)
            np_a = np_a + jnp.where(pos, 1, 0)
            i4 = idxL * 4
            rcx = plsc.load_gather(arm_v, [i4 + 0])
            rcy = plsc.load_gather(arm_v, [i4 + 1])
            rw = plsc.load_gather(arm_v, [i4 + 2])
            rh = plsc.load_gather(arm_v, [i4 + 3])
            mt = [plsc.load_gather(tgt_v, [i8 + j]) for j in range(4)]
            l0 = ((mt[0] + mt[2]) * 0.5 - rcx) / (_VAR0 * rw)
            l1 = ((mt[1] + mt[3]) * 0.5 - rcy) / (_VAR0 * rh)
            l2 = _log_pos((mt[2] - mt[0]) / rw) * (1.0 / _VAR1)
            l3 = _log_pos((mt[3] - mt[1]) / rh) * (1.0 / _VAR1)
            il4 = (lane16 + g * 16) * 4
            for j, lj in enumerate((l0, l1, l2, l3)):
                d = plsc.load_gather(loc_c[buf], [il4 + j]) - lj
                ad = jnp.abs(d)
                sl = jnp.where(ad < 1.0, 0.5 * d * d, ad - 0.5)
                ll_a = ll_a + jnp.where(pos, sl, 0.0)
            return ll_a, lcp_a, np_a
        return p2

    zf = jnp.zeros((16,), jnp.float32)
    acc = (zf, zf, jnp.zeros((16,), jnp.int32))
    for ch in range(_NCH):
        buf = ch % 2
        h_conf.wait()
        h_loc.wait()
        if ch + 1 < _NCH:
            h_conf = conf_dma(ch + 1, 1 - buf)
            h_loc = loc_dma(ch + 1, 1 - buf)
        acc = lax.fori_loop(0, _GPC, make_p2(ch, buf), acc)

    ll = jnp.sum(acc[0])
    lc_pos = jnp.sum(acc[1])
    npos = jnp.sum(acc[2])

    k = jnp.minimum(npos * _NEGPOS, _P - 1)

    def count_ge(cand):
        def cb(g, cnt):
            sb = plsc.bitcast(s_v[pl.ds(g * 16, 16)], jnp.int32)
            return cnt + jnp.sum((sb >= cand).astype(jnp.int32))
        return lax.fori_loop(0, _GROUPS, cb, jnp.int32(0))

    m = jnp.int32(0)
    for bit in range(30, -1, -1):
        cand = m | jnp.int32(1 << bit)
        m = jnp.where(count_ge(cand) >= k, cand, m)

    def cgt_ceq(g, c2):
        cg, ceq = c2
        sb = plsc.bitcast(s_v[pl.ds(g * 16, 16)], jnp.int32)
        return (cg + jnp.sum((sb > m).astype(jnp.int32)),
                ceq + jnp.sum((sb == m).astype(jnp.int32)))

    cnt_gt, cnt_eq = lax.fori_loop(0, _GROUPS, cgt_ceq,
                                   (jnp.int32(0), jnp.int32(0)))
    need_eq = k - cnt_gt

    def idx_search(_):
        def count_lt(cand):
            def cb(g, cnt):
                ds = pl.ds(g * 16, 16)
                sb = plsc.bitcast(s_v[ds], jnp.int32)
                idxL = lane16 + g * 16
                hit = (sb == m) & (idxL < cand)
                return cnt + jnp.sum(hit.astype(jnp.int32))
            return lax.fori_loop(0, _GROUPS, cb, jnp.int32(0))
        jt = jnp.int32(0)
        for bit in range(12, -1, -1):
            cand = jt | jnp.int32(1 << bit)
            jt = jnp.where(count_lt(cand) <= need_eq, cand, jt)
        return jt

    def idx_trivial(_):
        return jnp.where(need_eq > 0, jnp.int32(_PP), jnp.int32(0))

    partial_ties = (need_eq > 0) & (need_eq < cnt_eq)
    jthr = lax.cond(partial_ties, idx_search, idx_trivial, 0)

    v = plsc.bitcast(jnp.full((16,), m, jnp.int32), jnp.float32)

    def negsum(g, a):
        ds = pl.ds(g * 16, 16)
        sv = s_v[ds]
        idxL = lane16 + g * 16
        sel = (sv > v) | ((sv == v) & (idxL < jthr))
        return a + jnp.where(sel, sv, 0.0)

    lc = lc_pos + jnp.sum(lax.fori_loop(0, _GROUPS, negsum, zf))

    res = jnp.where(lane16 == 0, ll, 0.0)
    res = jnp.where(lane16 == 1, lc, res)
    res = jnp.where(lane16 == 2, npos.astype(jnp.float32), res)
    res_v[...] = res
    pltpu.sync_copy(res_v, out_h.at[pl.ds(b * 16, 16)])


@functools.partial(
    pl.kernel,
    out_type=jax.ShapeDtypeStruct((_B * 16,), jnp.float32),
    mesh=plsc.VectorSubcoreMesh(core_axis_name="c", subcore_axis_name="s"),
    compiler_params=pltpu.CompilerParams(needs_layout_passes=False, use_tc_tiling_on_sc=False),
    scratch_types=[
        pltpu.VMEM((_PP * 4,), jnp.float32),
        pltpu.VMEM((_PP * 4,), jnp.float32),
        pltpu.VMEM((_PP,), jnp.float32),
        pltpu.VMEM((_T * 8 + 16,), jnp.float32),
        pltpu.VMEM((_PP,), jnp.float32),
        pltpu.VMEM((_PP,), jnp.int32),
        pltpu.VMEM((_PP,), jnp.float32),
        pltpu.VMEM((_CW * _C,), jnp.float32),
        pltpu.VMEM((_CW * _C,), jnp.float32),
        pltpu.VMEM((_CW * 4,), jnp.float32),
        pltpu.VMEM((_CW * 4,), jnp.float32),
        pltpu.VMEM((16,), jnp.float32),
    ] + [pltpu.SemaphoreType.DMA] * 8,
)
def _sc_kernel(*args):
    (pri_h, arm_h, ac0_h, conf_h, locd_h, tgt_h, out_h,
     pri_v, arm_v, ac0_v, tgt_v, bto_v, bti_v, s_v,
     conf_c0, conf_c1, loc_c0, loc_c1, res_v, *sems) = args
    _sc_body(pri_h, arm_h, ac0_h, conf_h, locd_h, tgt_h, out_h,
             pri_v, arm_v, ac0_v, tgt_v, bto_v, bti_v, s_v,
             (conf_c0, conf_c1), (loc_c0, loc_c1), res_v, tuple(sems))


def kernel(arm_loc, arm_conf, loc_data, conf_data, priors, targets):
    padp = _PP - _P
    pri = jnp.pad(priors, ((0, padp), (0, 0))).reshape(_PP * 4)
    arm = jnp.pad(arm_loc, ((0, 0), (0, padp), (0, 0))).reshape(_B * _PP * 4)
    ac0 = jnp.pad(arm_conf[:, :, 0], ((0, 0), (0, padp)),
                  constant_values=2.0).reshape(_B * _PP)
    conf = jnp.pad(conf_data, ((0, 0), (0, padp), (0, 0)))
    conf = conf.reshape(_B * _NCH * _CW * _C)
    locd = jnp.pad(loc_data, ((0, 0), (0, padp), (0, 0)))
    locd = locd.reshape(_B * _NCH * _CW * 4)
    tgt = jnp.pad(targets, ((0, 0), (0, 0), (0, 3))).reshape(_B, _T * 8)
    tgt = jnp.pad(tgt, ((0, 0), (0, 16))).reshape(_B * 96)

    out = _sc_kernel(pri, arm, ac0, conf, locd, tgt).reshape(_B, 16)
    n = jnp.sum(out[:, 2])
    return jnp.sum(out[:, 0]) / n, jnp.sum(out[:, 1]) / n
